# Initial kernel scaffold; baseline (speedup 1.0000x reference)
import jax
import jax.numpy as jnp
from jax.experimental import pallas as pl


def kernel(x, edge_index, W1, b1, W2, b2):
    raise NotImplementedError("write your pallas kernel here")


# speedup vs baseline: 21.9088x; 21.9088x over previous
import functools

import jax
import jax.numpy as jnp
from jax import lax
from jax.experimental import pallas as pl
from jax.experimental.pallas import tpu as pltpu
from jax.experimental.pallas import tpu_sc as plsc

_N = 10000
_E = 320000
_D_IN = 128
_D_HID = 16
_D_OUT = 40

_NC = 2
_NS = 16
_NW = _NC * _NS
_EPW = _E // _NW
_CH = 80
_NCH = _EPW // _CH
_RPS = _N // _NS

_BLK = 1000
_NBLK = _N // _BLK


def _prop_body(g_hbm, row_hbm, col_hbm, out_hbm,
               row_v, col_v, rows_v, acc_sh, sem):
    c = lax.axis_index("c")
    s = lax.axis_index("s")
    wid = s * _NC + c

    pltpu.sync_copy(g_hbm.at[pl.ds(s * _RPS, _RPS)],
                    acc_sh.at[pl.ds(s * _RPS, _RPS)])

    pltpu.sync_copy(row_hbm.at[wid], row_v)
    pltpu.sync_copy(col_hbm.at[wid], col_v)
    plsc.subcore_barrier()

    def _chunk(j, carry):
        pltpu.async_copy(g_hbm.at[row_v.at[j]], rows_v, sem).wait()
        pltpu.sync_copy(rows_v, acc_sh.at[col_v.at[j]], add=True)
        return carry
    lax.fori_loop(0, _NCH, _chunk, 0)
    plsc.subcore_barrier()

    pltpu.sync_copy(acc_sh.at[pl.ds(s * _RPS, _RPS)],
                    out_hbm.at[c, pl.ds(s * _RPS, _RPS)])


_prop = functools.partial(
    pl.kernel,
    out_type=jax.ShapeDtypeStruct((_NC, _N, _D_HID), jnp.float32),
    scratch_types=[
        pltpu.VMEM((_NCH, _CH), jnp.int32),
        pltpu.VMEM((_NCH, _CH), jnp.int32),
        pltpu.VMEM((_CH, _D_HID), jnp.float32),
        pltpu.VMEM_SHARED((_N, _D_HID), jnp.float32),
        pltpu.SemaphoreType.DMA,
    ],
    mesh=plsc.VectorSubcoreMesh(core_axis_name="c", subcore_axis_name="s"),
    compiler_params=pltpu.CompilerParams(use_tc_tiling_on_sc=False),
)(_prop_body)


def _lin1_body(x_ref, w_ref, dp0_ref, dp1_ref, g_ref, dv_ref):
    dv = lax.rsqrt(dp0_ref[...] + dp1_ref[...] - 1.0)
    h = jnp.dot(x_ref[...], w_ref[...], preferred_element_type=jnp.float32)
    g_ref[...] = h * dv
    dv_ref[...] = dv


def _mid_body(p0_ref, p1_ref, g1_ref, dv_ref, b1_ref, g2_ref):
    s = dv_ref[...] * (p0_ref[...] + p1_ref[...] - g1_ref[...])
    g2_ref[...] = dv_ref[...] * jnp.maximum(s + b1_ref[...], 0.0)


def _fin_body(q0_ref, q1_ref, g2_ref, dv_ref, w2_ref, b2_ref, out_ref):
    s = dv_ref[...] * (q0_ref[...] + q1_ref[...] - g2_ref[...])
    out_ref[...] = (
        jnp.dot(s, w2_ref[...], preferred_element_type=jnp.float32)
        + b2_ref[...]
    )


def _row_blk(d):
    return pl.BlockSpec((_BLK, d), lambda i: (i, 0))


def _full(shape):
    return pl.BlockSpec(shape, lambda i: (0, 0))


_lin1 = pl.pallas_call(
    _lin1_body,
    grid=(_NBLK,),
    in_specs=[_row_blk(_D_IN), _full((_D_IN, _D_HID)),
              _row_blk(_D_HID), _row_blk(_D_HID)],
    out_specs=[_row_blk(_D_HID), _row_blk(_D_HID)],
    out_shape=[jax.ShapeDtypeStruct((_N, _D_HID), jnp.float32),
               jax.ShapeDtypeStruct((_N, _D_HID), jnp.float32)],
)

_mid = pl.pallas_call(
    _mid_body,
    grid=(_NBLK,),
    in_specs=[_row_blk(_D_HID), _row_blk(_D_HID), _row_blk(_D_HID),
              _row_blk(_D_HID), _full((1, _D_HID))],
    out_specs=_row_blk(_D_HID),
    out_shape=jax.ShapeDtypeStruct((_N, _D_HID), jnp.float32),
)

_fin = pl.pallas_call(
    _fin_body,
    grid=(_NBLK,),
    in_specs=[_row_blk(_D_HID), _row_blk(_D_HID), _row_blk(_D_HID),
              _row_blk(_D_HID), _full((_D_HID, _D_OUT)), _full((1, _D_OUT))],
    out_specs=_row_blk(_D_OUT),
    out_shape=jax.ShapeDtypeStruct((_N, _D_OUT), jnp.float32),
)


def kernel(x, edge_index, W1, b1, W2, b2):
    row = edge_index[0].astype(jnp.int32).reshape(_NW, _NCH, _CH)
    col = edge_index[1].astype(jnp.int32).reshape(_NW, _NCH, _CH)

    ones_tab = jnp.ones((_N, _D_HID), dtype=jnp.float32)
    dp = _prop(ones_tab, row, col)

    g1, dv = _lin1(x, W1, dp[0], dp[1])

    p = _prop(g1, row, col)
    g2 = _mid(p[0], p[1], g1, dv, b1.reshape(1, _D_HID))

    q = _prop(g2, row, col)
    out = _fin(q[0], q[1], g2, dv, W2, b2.reshape(1, _D_OUT))
    return out


# speedup vs baseline: 39.9227x; 1.8222x over previous
import functools

import jax
import jax.numpy as jnp
from jax import lax
from jax.experimental import pallas as pl
from jax.experimental.pallas import tpu as pltpu
from jax.experimental.pallas import tpu_sc as plsc

_N = 10000
_E = 320000
_D_IN = 128
_D_HID = 16
_D_OUT = 40

_NC = 2
_NS = 16
_NW = _NC * _NS
_CH = 128
_NCHK = 80
_EPW = _CH * _NCHK
_EPAD = _EPW * _NW
_RPS = _N // _NS
_NPAD = _N + 16
_NBUF = 4

_BLK = 1000
_NBLK = _N // _BLK

_SC_PARAMS = pltpu.CompilerParams(use_tc_tiling_on_sc=False)
_MESH = plsc.VectorSubcoreMesh(core_axis_name="c", subcore_axis_name="s")


def _prop_body(g_hbm, row_hbm, col_hbm, out_hbm,
               row_v, col_v, rows_a, rows_b, rows_c, rows_d,
               acc_sh, sem_a, sem_b, sem_c, sem_d):
    c = lax.axis_index("c")
    s = lax.axis_index("s")
    wid = s * _NC + c
    bufs = (rows_a, rows_b, rows_c, rows_d)
    sems = (sem_a, sem_b, sem_c, sem_d)

    pltpu.sync_copy(g_hbm.at[pl.ds(s * _RPS, _RPS)],
                    acc_sh.at[pl.ds(s * _RPS, _RPS)])

    pltpu.sync_copy(row_hbm.at[wid], row_v)
    pltpu.sync_copy(col_hbm.at[wid], col_v)
    plsc.subcore_barrier()

    for b in range(_NBUF):
        pltpu.async_copy(g_hbm.at[row_v.at[b]], bufs[b], sems[b])

    def _block(i, carry):
        j0 = i * _NBUF
        for b in range(_NBUF):
            j = j0 + b
            pltpu.make_async_copy(g_hbm.at[row_v.at[j]], bufs[b],
                                  sems[b]).wait()
            pltpu.sync_copy(bufs[b], acc_sh.at[col_v.at[j]], add=True)

            @pl.when(j + _NBUF < _NCHK)
            def _(b=b, j=j):
                pltpu.async_copy(g_hbm.at[row_v.at[j + _NBUF]], bufs[b],
                                 sems[b])
        return carry

    lax.fori_loop(0, _NCHK // _NBUF, _block, 0)
    plsc.subcore_barrier()

    pltpu.sync_copy(acc_sh.at[pl.ds(s * _RPS, _RPS)],
                    out_hbm.at[c, pl.ds(s * _RPS, _RPS)])


_prop = functools.partial(
    pl.kernel,
    out_type=jax.ShapeDtypeStruct((_NC, _N, _D_HID), jnp.float32),
    scratch_types=[
        pltpu.VMEM((_NCHK, _CH), jnp.int32),
        pltpu.VMEM((_NCHK, _CH), jnp.int32),
        pltpu.VMEM((_CH, _D_HID), jnp.float32),
        pltpu.VMEM((_CH, _D_HID), jnp.float32),
        pltpu.VMEM((_CH, _D_HID), jnp.float32),
        pltpu.VMEM((_CH, _D_HID), jnp.float32),
        pltpu.VMEM_SHARED((_NPAD, _D_HID), jnp.float32),
        pltpu.SemaphoreType.DMA,
        pltpu.SemaphoreType.DMA,
        pltpu.SemaphoreType.DMA,
        pltpu.SemaphoreType.DMA,
    ],
    mesh=_MESH,
    compiler_params=_SC_PARAMS,
)(_prop_body)


def _deg_body(ones_hbm, col_hbm, out_hbm, col_v, ones_v, acc_sh, sem):
    c = lax.axis_index("c")
    s = lax.axis_index("s")
    wid = s * _NC + c

    @pl.when(s == 0)
    def _():
        pltpu.sync_copy(ones_hbm, acc_sh.at[pl.ds(0, _N)])

    for k in range(_CH // 16):
        ones_v[pl.ds(k * 16, 16)] = jnp.ones((16,), jnp.float32)
    pltpu.sync_copy(col_hbm.at[wid], col_v)
    plsc.subcore_barrier()

    def _chunk(j, carry):
        pltpu.sync_copy(ones_v, acc_sh.at[col_v.at[j]], add=True)
        return carry

    lax.fori_loop(0, _NCHK, _chunk, 0)
    plsc.subcore_barrier()

    @pl.when(s == 0)
    def _():
        pltpu.sync_copy(acc_sh.at[pl.ds(0, _N)], out_hbm.at[c])


_deg = functools.partial(
    pl.kernel,
    out_type=jax.ShapeDtypeStruct((_NC, _N), jnp.float32),
    scratch_types=[
        pltpu.VMEM((_NCHK, _CH), jnp.int32),
        pltpu.VMEM((_CH,), jnp.float32),
        pltpu.VMEM_SHARED((_NPAD,), jnp.float32),
        pltpu.SemaphoreType.DMA,
    ],
    mesh=_MESH,
    compiler_params=_SC_PARAMS,
)(_deg_body)


def _lin1_body(x_ref, w_ref, dp0_ref, dp1_ref, g_ref, dv_ref):
    dv = lax.rsqrt(dp0_ref[...] + dp1_ref[...] - 1.0)
    h = jnp.dot(x_ref[...], w_ref[...], preferred_element_type=jnp.float32)
    g_ref[...] = h * dv
    dv_ref[...] = dv


def _mid_body(p0_ref, p1_ref, g1_ref, dv_ref, b1_ref, g2_ref):
    s = dv_ref[...] * (p0_ref[...] + p1_ref[...] - g1_ref[...])
    g2_ref[...] = dv_ref[...] * jnp.maximum(s + b1_ref[...], 0.0)


def _fin_body(q0_ref, q1_ref, g2_ref, dv_ref, w2_ref, b2_ref, out_ref):
    s = dv_ref[...] * (q0_ref[...] + q1_ref[...] - g2_ref[...])
    out_ref[...] = (
        jnp.dot(s, w2_ref[...], preferred_element_type=jnp.float32)
        + b2_ref[...]
    )


def _row_blk(d):
    return pl.BlockSpec((_BLK, d), lambda i: (i, 0))


def _full(shape):
    return pl.BlockSpec(shape, lambda i: (0, 0))


_lin1 = pl.pallas_call(
    _lin1_body,
    grid=(_NBLK,),
    in_specs=[_row_blk(_D_IN), _full((_D_IN, _D_HID)),
              _row_blk(1), _row_blk(1)],
    out_specs=[_row_blk(_D_HID), _row_blk(1)],
    out_shape=[jax.ShapeDtypeStruct((_N, _D_HID), jnp.float32),
               jax.ShapeDtypeStruct((_N, 1), jnp.float32)],
)

_mid = pl.pallas_call(
    _mid_body,
    grid=(_NBLK,),
    in_specs=[_row_blk(_D_HID), _row_blk(_D_HID), _row_blk(_D_HID),
              _row_blk(1), _full((1, _D_HID))],
    out_specs=_row_blk(_D_HID),
    out_shape=jax.ShapeDtypeStruct((_N, _D_HID), jnp.float32),
)

_fin = pl.pallas_call(
    _fin_body,
    grid=(_NBLK,),
    in_specs=[_row_blk(_D_HID), _row_blk(_D_HID), _row_blk(_D_HID),
              _row_blk(1), _full((_D_HID, _D_OUT)), _full((1, _D_OUT))],
    out_specs=_row_blk(_D_OUT),
    out_shape=jax.ShapeDtypeStruct((_N, _D_OUT), jnp.float32),
)


def kernel(x, edge_index, W1, b1, W2, b2):
    row = edge_index[0].astype(jnp.int32)
    col = edge_index[1].astype(jnp.int32)
    rowp = jnp.concatenate(
        [row, jnp.zeros((_EPAD - _E,), jnp.int32)]).reshape(_NW, _NCHK, _CH)
    colp = jnp.concatenate(
        [col, jnp.full((_EPAD - _E,), _N, jnp.int32)]).reshape(_NW, _NCHK, _CH)

    ones_n = jnp.ones((_N,), dtype=jnp.float32)
    dp = _deg(ones_n, colp)
    dp0 = dp[0].reshape(_N, 1)
    dp1 = dp[1].reshape(_N, 1)

    g1, dv = _lin1(x, W1, dp0, dp1)

    p = _prop(g1, rowp, colp)
    g2 = _mid(p[0], p[1], g1, dv, b1.reshape(1, _D_HID))

    q = _prop(g2, rowp, colp)
    out = _fin(q[0], q[1], g2, dv, W2, b2.reshape(1, _D_OUT))
    return out
